# baseline (speedup 1.0000x reference)
import functools

import jax
import jax.numpy as jnp
from jax import lax
from jax.experimental import pallas as pl
from jax.experimental.pallas import tpu as pltpu
from jax.experimental.pallas import tpu_sc as plsc

EPS = 1e-5
BLK = 1088


def _pad_rows(n):
    return -(-n // 128) * 128


def _proj_body(x4_ref, win4_ref, wl_ref, wh_ref, lo_ref, hi_ref, w4l, w4h):
    f32 = jnp.float32

    @pl.when(pl.program_id(0) == 0)
    def _():
        win4 = win4_ref[...]
        w4l[...] = jnp.dot(win4, wl_ref[...], preferred_element_type=f32)
        w4h[...] = jnp.dot(win4, wh_ref[...], preferred_element_type=f32)

    x4 = x4_ref[...]
    lo_ref[...] = jnp.dot(x4, w4l[...], preferred_element_type=f32)
    hi_ref[...] = jnp.dot(x4, w4h[...], preferred_element_type=f32)


def _proj(x, win, w2):
    n, d_in = x.shape
    h = win.shape[1]
    hh = h // 2
    np_ = _pad_rows(n)
    f32 = jnp.float32
    x4 = x.reshape(n // 4, 4 * d_in)
    eye4 = jnp.eye(4, dtype=f32)
    win4 = jnp.kron(eye4, win)
    wl = jnp.kron(eye4, w2[:, :hh])
    wh = jnp.kron(eye4, w2[:, hh:])
    out = jax.ShapeDtypeStruct((np_ // 4, 128), f32)
    wspec = pl.BlockSpec((4 * h, 128), lambda i: (0, 0))
    return pl.pallas_call(
        _proj_body,
        grid=(np_ // BLK,),
        in_specs=[
            pl.BlockSpec((BLK // 4, 4 * d_in), lambda i: (i, 0)),
            pl.BlockSpec((4 * d_in, 4 * h), lambda i: (0, 0)),
            wspec, wspec,
        ],
        out_specs=[pl.BlockSpec((BLK // 4, 128), lambda i: (i, 0))] * 2,
        out_shape=[out] * 2,
        scratch_shapes=[pltpu.VMEM((4 * d_in, 128), f32)] * 2,
    )(x4, win4, wl, wh)


NR = 4
NI = 8


def _make_segsum(n, e):
    np_ = _pad_rows(n)
    ns = 16
    rpt = np_ // ns
    ept = e // ns
    ch = 128
    nfull = ept // ch
    tail = ept - nfull * ch
    assert np_ % (8 * ns) == 0 and e % ns == 0 and tail % 8 == 0
    assert nfull >= 3 * NI
    loop_lo = NI
    loop_hi = loop_lo + ((nfull - loop_lo - (NI - NR)) // NI) * NI
    nblocks = (loop_hi - loop_lo) // NI
    n_peel_hi = nfull - loop_hi

    mesh = plsc.VectorSubcoreMesh(core_axis_name="c", subcore_axis_name="s")
    half = jax.ShapeDtypeStruct((np_, 32), jnp.float32)
    scratch = (
        [pltpu.VMEM_SHARED((np_, 32), jnp.float32)]
        + [pltpu.VMEM((ch,), jnp.int32) for _ in range(2 * NI)]
        + [pltpu.VMEM((ch, 32), jnp.float32) for _ in range(NR)]
        + [pltpu.VMEM((tail,), jnp.int32) for _ in range(2)]
        + [pltpu.VMEM((tail, 32), jnp.float32)]
        + [pltpu.SemaphoreType.DMA for _ in range(NI + 2 * NR + 1)]
    )

    @functools.partial(
        pl.kernel,
        out_type=(half, half),
        mesh=mesh,
        scratch_types=scratch,
        compiler_params=pltpu.CompilerParams(use_tc_tiling_on_sc=False),
    )
    def seg(glo, ghi, zeros, ei, out_lo, out_hi, acc, *sc):
        sidx = sc[0:NI]
        didx = sc[NI:2 * NI]
        rows = sc[2 * NI:2 * NI + NR]
        sidxt, didxt, rowst = sc[2 * NI + NR:2 * NI + NR + 3]
        sems = sc[2 * NI + NR + 3:]
        semi = sems[0:NI]
        semg = sems[NI:NI + NR]
        semsc = sems[NI + NR:NI + 2 * NR]
        semt = sems[NI + 2 * NR]

        c = lax.axis_index("c")
        s = lax.axis_index("s")
        r0 = s * rpt
        e0 = s * ept

        def run(g, out):
            def idx_start(i, bi):
                eb = e0 + i * ch
                pltpu.async_copy(ei.at[0, pl.ds(eb, ch)], sidx[bi], semi[bi])
                pltpu.async_copy(ei.at[1, pl.ds(eb, ch)], didx[bi], semi[bi])

            def idx_wait(bi):
                pltpu.make_async_copy(ei.at[0, pl.ds(0, ch)], sidx[bi], semi[bi]).wait()
                pltpu.make_async_copy(ei.at[1, pl.ds(0, ch)], didx[bi], semi[bi]).wait()

            def g_start(br, bi):
                pltpu.async_copy(g.at[sidx[bi]], rows[br], semg[br])

            def g_wait(br, bi):
                pltpu.make_async_copy(g.at[sidx[bi]], rows[br], semg[br]).wait()

            def s_start(br, bi):
                pltpu.async_copy(rows[br], acc.at[didx[bi]], semsc[br], add=True)

            def s_wait(br, bi):
                pltpu.make_async_copy(rows[br], acc.at[didx[bi]], semsc[br]).wait()

            def visit(i, j, do_swait, do_gwait, do_idx):
                br, bi = j % NR, j % NI
                if do_swait:
                    s_wait(br, (j - NR) % NI)
                idx_wait(bi)
                g_start(br, bi)
                if do_gwait:
                    g_wait((j - 1) % NR, (j - 1) % NI)
                    s_start((j - 1) % NR, (j - 1) % NI)
                if do_idx:
                    idx_start(i + (NI - NR), (j + (NI - NR)) % NI)

            pltpu.sync_copy(zeros.at[pl.ds(r0, rpt)], acc.at[pl.ds(r0, rpt)])
            for j in range(NI - NR):
                idx_start(j, j)
            plsc.subcore_barrier()

            for j in range(NI):
                visit(j, j, do_swait=(j >= NR), do_gwait=(j >= 1), do_idx=True)

            def outer(k, carry):
                o = k * NI
                for j in range(NI):
                    visit(o + j, j, True, True, True)
                return carry

            if nblocks > 0:
                lax.fori_loop(1, 1 + nblocks, outer, 0)

            for jj in range(n_peel_hi):
                i = loop_hi + jj
                visit(i, i % NI, True, True, do_idx=(i + (NI - NR) < nfull))

            last = nfull - 1
            g_wait(last % NR, last % NI)
            s_start(last % NR, last % NI)
            if tail:
                et = e0 + nfull * ch
                pltpu.sync_copy(ei.at[0, pl.ds(et, tail)], sidxt)
                pltpu.sync_copy(ei.at[1, pl.ds(et, tail)], didxt)
                pltpu.async_copy(g.at[sidxt], rowst, semt).wait()
                pltpu.async_copy(rowst, acc.at[didxt], semt, add=True)
            for d in range(NR - 1, -1, -1):
                cb = last - d
                s_wait(cb % NR, cb % NI)
            if tail:
                pltpu.make_async_copy(rowst, acc.at[didxt], semt).wait()
            plsc.subcore_barrier()
            pltpu.sync_copy(acc.at[pl.ds(r0, rpt)], out.at[pl.ds(r0, rpt)])

        @pl.when(c == 0)
        def _():
            run(glo, out_lo)

        @pl.when(c == 1)
        def _():
            run(ghi, out_hi)

    return seg


def _post_body(mlo_ref, mhi_ref, rlo_ref, rhi_ref, brl_ref, brh_ref,
               gl_ref, gh_ref, bl_ref, bh_ref, gsum_ref, gbc_ref,
               wexl_ref, wexh_ref, bout4_ref, y4_ref):
    f32 = jnp.float32
    a = mlo_ref[...] + rlo_ref[...] + brl_ref[...]
    b = mhi_ref[...] + rhi_ref[...] + brh_ref[...]
    gsum = gsum_ref[...]
    gbc = gbc_ref[...]
    ssum = (jnp.dot(a, gsum, preferred_element_type=f32)
            + jnp.dot(b, gsum, preferred_element_type=f32))
    mu = jnp.dot(ssum / 64.0, gbc, preferred_element_type=f32)
    da = a - mu
    db = b - mu
    vsum = (jnp.dot(da * da, gsum, preferred_element_type=f32)
            + jnp.dot(db * db, gsum, preferred_element_type=f32))
    var = jnp.dot(vsum / 64.0, gbc, preferred_element_type=f32)
    rs = lax.rsqrt(var + EPS)
    va = jnp.maximum(da * rs * gl_ref[...] + bl_ref[...], 0.0)
    vb = jnp.maximum(db * rs * gh_ref[...] + bh_ref[...], 0.0)
    y4_ref[...] = (jnp.dot(va, wexl_ref[...], preferred_element_type=f32)
                   + jnp.dot(vb, wexh_ref[...], preferred_element_type=f32)
                   + bout4_ref[...])


def _post(m_lo, m_hi, r_lo, r_hi, n, brel, ln_g, ln_b, wout, bout):
    h = brel.shape[0]
    hh = h // 2
    d_out = wout.shape[1]
    np_ = _pad_rows(n)
    f32 = jnp.float32
    mlo4 = m_lo.reshape(np_ // 4, 128)
    mhi4 = m_hi.reshape(np_ // 4, 128)
    brl = jnp.tile(brel[:hh], 4).reshape(1, 128)
    brh = jnp.tile(brel[hh:], 4).reshape(1, 128)
    gl = jnp.tile(ln_g[:hh], 4).reshape(1, 128)
    gh = jnp.tile(ln_g[hh:], 4).reshape(1, 128)
    bl = jnp.tile(ln_b[:hh], 4).reshape(1, 128)
    bh = jnp.tile(ln_b[hh:], 4).reshape(1, 128)
    bout4 = jnp.tile(bout, 4).reshape(1, 4 * d_out)
    gi = jnp.arange(128, dtype=jnp.int32) // hh
    gsum = (gi[:, None] == jnp.arange(4)[None, :]).astype(f32)
    gbc = gsum.T
    eye4 = jnp.eye(4, dtype=f32)
    wexl = jnp.kron(eye4, wout[:hh, :])
    wexh = jnp.kron(eye4, wout[hh:, :])
    cspec = pl.BlockSpec((1, 128), lambda i: (0, 0))
    y4 = pl.pallas_call(
        _post_body,
        grid=(np_ // BLK,),
        in_specs=[
            pl.BlockSpec((BLK // 4, 128), lambda i: (i, 0)),
            pl.BlockSpec((BLK // 4, 128), lambda i: (i, 0)),
            pl.BlockSpec((BLK // 4, 128), lambda i: (i, 0)),
            pl.BlockSpec((BLK // 4, 128), lambda i: (i, 0)),
            cspec, cspec, cspec, cspec, cspec, cspec,
            pl.BlockSpec((128, 4), lambda i: (0, 0)),
            pl.BlockSpec((4, 128), lambda i: (0, 0)),
            pl.BlockSpec((128, 4 * d_out), lambda i: (0, 0)),
            pl.BlockSpec((128, 4 * d_out), lambda i: (0, 0)),
            pl.BlockSpec((1, 4 * d_out), lambda i: (0, 0)),
        ],
        out_specs=pl.BlockSpec((BLK // 4, 4 * d_out), lambda i: (i, 0)),
        out_shape=jax.ShapeDtypeStruct((n // 4, 4 * d_out), f32),
    )(mlo4, mhi4, r_lo, r_hi, brl, brh, gl, gh, bl, bh, gsum, gbc,
      wexl, wexh, bout4)
    return y4.reshape(n, d_out)


def kernel(x_user, x_item, edge_index_user_clicks_item, edge_index_item_rev_clicks_user,
           Win_user, Win_item, Wrel_uc, brel_uc, Wroot_uc, Wrel_iu, brel_iu, Wroot_iu,
           ln_g_user, ln_b_user, ln_g_item, ln_b_item,
           Wout_user, bout_user, Wout_item, bout_item):
    n_user = x_user.shape[0]
    n_item = x_item.shape[0]
    e_uc = edge_index_user_clicks_item.shape[1]
    e_iu = edge_index_item_rev_clicks_user.shape[1]
    np_u = _pad_rows(n_user)
    np_i = _pad_rows(n_item)

    gu_lo, gu_hi = _proj(x_user, Win_user, Wrel_uc)
    gi_lo, gi_hi = _proj(x_item, Win_item, Wrel_iu)
    ru_lo, ru_hi = _proj(x_user, Win_user, Wroot_iu)
    ri_lo, ri_hi = _proj(x_item, Win_item, Wroot_uc)

    zeros_u = jnp.zeros((np_u, 32), jnp.float32)
    zeros_i = zeros_u if np_i == np_u else jnp.zeros((np_i, 32), jnp.float32)

    seg_uc = _make_segsum(n_item, e_uc)
    seg_iu = _make_segsum(n_user, e_iu)
    def lin(a):
        return a.reshape(a.shape[0] * 4, 32)

    mi_lo, mi_hi = seg_uc(lin(gu_lo), lin(gu_hi), zeros_i, edge_index_user_clicks_item)
    mu_lo, mu_hi = seg_iu(lin(gi_lo), lin(gi_hi), zeros_u, edge_index_item_rev_clicks_user)

    y_item = _post(mi_lo, mi_hi, ri_lo, ri_hi, n_item, brel_uc,
                   ln_g_item, ln_b_item, Wout_item, bout_item)
    y_user = _post(mu_lo, mu_hi, ru_lo, ru_hi, n_user, brel_iu,
                   ln_g_user, ln_b_user, Wout_user, bout_user)
    return (y_user, y_item)


# speedup vs baseline: 1.0139x; 1.0139x over previous
import functools

import jax
import jax.numpy as jnp
from jax import lax
from jax.experimental import pallas as pl
from jax.experimental.pallas import tpu as pltpu
from jax.experimental.pallas import tpu_sc as plsc

EPS = 1e-5
BLK = 1088


def _pad_rows(n):
    return -(-n // 128) * 128


def _proj_body(x4_ref, win4_ref, wl_ref, wh_ref, lo_ref, hi_ref, w4l, w4h):
    f32 = jnp.float32

    @pl.when(pl.program_id(0) == 0)
    def _():
        win4 = win4_ref[...]
        w4l[...] = jnp.dot(win4, wl_ref[...], preferred_element_type=f32)
        w4h[...] = jnp.dot(win4, wh_ref[...], preferred_element_type=f32)

    x4 = x4_ref[...]
    lo_ref[...] = jnp.dot(x4, w4l[...], preferred_element_type=f32)
    hi_ref[...] = jnp.dot(x4, w4h[...], preferred_element_type=f32)


def _proj(x, win, w2):
    n, d_in = x.shape
    h = win.shape[1]
    hh = h // 2
    np_ = _pad_rows(n)
    f32 = jnp.float32
    x4 = x.reshape(n // 4, 4 * d_in)
    eye4 = jnp.eye(4, dtype=f32)
    win4 = jnp.kron(eye4, win)
    wl = jnp.kron(eye4, w2[:, :hh])
    wh = jnp.kron(eye4, w2[:, hh:])
    out = jax.ShapeDtypeStruct((np_ // 4, 128), f32)
    wspec = pl.BlockSpec((4 * h, 128), lambda i: (0, 0))
    return pl.pallas_call(
        _proj_body,
        grid=(np_ // BLK,),
        in_specs=[
            pl.BlockSpec((BLK // 4, 4 * d_in), lambda i: (i, 0)),
            pl.BlockSpec((4 * d_in, 4 * h), lambda i: (0, 0)),
            wspec, wspec,
        ],
        out_specs=[pl.BlockSpec((BLK // 4, 128), lambda i: (i, 0))] * 2,
        out_shape=[out] * 2,
        scratch_shapes=[pltpu.VMEM((4 * d_in, 128), f32)] * 2,
    )(x4, win4, wl, wh)


NR = 3
NI = 6


def _make_segsum(n, e):
    np_ = _pad_rows(n)
    ns = 16
    rpt = np_ // ns
    ept = e // ns
    ch = 128
    nfull = ept // ch
    tail = ept - nfull * ch
    assert np_ % (8 * ns) == 0 and e % ns == 0 and tail % 8 == 0
    assert nfull >= 3 * NI
    loop_lo = NI
    loop_hi = loop_lo + ((nfull - loop_lo - (NI - NR)) // NI) * NI
    nblocks = (loop_hi - loop_lo) // NI
    n_peel_hi = nfull - loop_hi

    mesh = plsc.VectorSubcoreMesh(core_axis_name="c", subcore_axis_name="s")
    half = jax.ShapeDtypeStruct((np_, 32), jnp.float32)
    scratch = (
        [pltpu.VMEM_SHARED((np_, 32), jnp.float32)]
        + [pltpu.VMEM((ch,), jnp.int32) for _ in range(2 * NI)]
        + [pltpu.VMEM((ch, 32), jnp.float32) for _ in range(NR)]
        + [pltpu.VMEM((tail,), jnp.int32) for _ in range(2)]
        + [pltpu.VMEM((tail, 32), jnp.float32)]
        + [pltpu.SemaphoreType.DMA for _ in range(NI + 2 * NR + 1)]
    )

    @functools.partial(
        pl.kernel,
        out_type=(half, half),
        mesh=mesh,
        scratch_types=scratch,
        compiler_params=pltpu.CompilerParams(use_tc_tiling_on_sc=False),
    )
    def seg(glo, ghi, zeros, ei, out_lo, out_hi, acc, *sc):
        sidx = sc[0:NI]
        didx = sc[NI:2 * NI]
        rows = sc[2 * NI:2 * NI + NR]
        sidxt, didxt, rowst = sc[2 * NI + NR:2 * NI + NR + 3]
        sems = sc[2 * NI + NR + 3:]
        semi = sems[0:NI]
        semg = sems[NI:NI + NR]
        semsc = sems[NI + NR:NI + 2 * NR]
        semt = sems[NI + 2 * NR]

        c = lax.axis_index("c")
        s = lax.axis_index("s")
        r0 = s * rpt
        e0 = s * ept

        def run(g, out):
            def idx_start(i, bi):
                eb = e0 + i * ch
                pltpu.async_copy(ei.at[0, pl.ds(eb, ch)], sidx[bi], semi[bi])
                pltpu.async_copy(ei.at[1, pl.ds(eb, ch)], didx[bi], semi[bi])

            def idx_wait(bi):
                pltpu.make_async_copy(ei.at[0, pl.ds(0, ch)], sidx[bi], semi[bi]).wait()
                pltpu.make_async_copy(ei.at[1, pl.ds(0, ch)], didx[bi], semi[bi]).wait()

            def g_start(br, bi):
                pltpu.async_copy(g.at[sidx[bi]], rows[br], semg[br])

            def g_wait(br, bi):
                pltpu.make_async_copy(g.at[sidx[bi]], rows[br], semg[br]).wait()

            def s_start(br, bi):
                pltpu.async_copy(rows[br], acc.at[didx[bi]], semsc[br], add=True)

            def s_wait(br, bi):
                pltpu.make_async_copy(rows[br], acc.at[didx[bi]], semsc[br]).wait()

            def visit(i, j, do_swait, do_gwait, do_idx):
                br, bi = j % NR, j % NI
                if do_swait:
                    s_wait(br, (j - NR) % NI)
                idx_wait(bi)
                g_start(br, bi)
                if do_gwait:
                    g_wait((j - 1) % NR, (j - 1) % NI)
                    s_start((j - 1) % NR, (j - 1) % NI)
                if do_idx:
                    idx_start(i + (NI - NR), (j + (NI - NR)) % NI)

            pltpu.sync_copy(zeros.at[pl.ds(r0, rpt)], acc.at[pl.ds(r0, rpt)])
            for j in range(NI - NR):
                idx_start(j, j)
            plsc.subcore_barrier()

            for j in range(NI):
                visit(j, j, do_swait=(j >= NR), do_gwait=(j >= 1), do_idx=True)

            def outer(k, carry):
                o = k * NI
                for j in range(NI):
                    visit(o + j, j, True, True, True)
                return carry

            if nblocks > 0:
                lax.fori_loop(1, 1 + nblocks, outer, 0)

            for jj in range(n_peel_hi):
                i = loop_hi + jj
                visit(i, i % NI, True, True, do_idx=(i + (NI - NR) < nfull))

            last = nfull - 1
            g_wait(last % NR, last % NI)
            s_start(last % NR, last % NI)
            if tail:
                et = e0 + nfull * ch
                pltpu.sync_copy(ei.at[0, pl.ds(et, tail)], sidxt)
                pltpu.sync_copy(ei.at[1, pl.ds(et, tail)], didxt)
                pltpu.async_copy(g.at[sidxt], rowst, semt).wait()
                pltpu.async_copy(rowst, acc.at[didxt], semt, add=True)
            for d in range(NR - 1, -1, -1):
                cb = last - d
                s_wait(cb % NR, cb % NI)
            if tail:
                pltpu.make_async_copy(rowst, acc.at[didxt], semt).wait()
            plsc.subcore_barrier()
            pltpu.sync_copy(acc.at[pl.ds(r0, rpt)], out.at[pl.ds(r0, rpt)])

        @pl.when(c == 0)
        def _():
            run(glo, out_lo)

        @pl.when(c == 1)
        def _():
            run(ghi, out_hi)

    return seg


def _post_body(mlo_ref, mhi_ref, rlo_ref, rhi_ref, brl_ref, brh_ref,
               gl_ref, gh_ref, bl_ref, bh_ref, gsum_ref, gbc_ref,
               wexl_ref, wexh_ref, bout4_ref, y4_ref):
    f32 = jnp.float32
    a = mlo_ref[...] + rlo_ref[...] + brl_ref[...]
    b = mhi_ref[...] + rhi_ref[...] + brh_ref[...]
    gsum = gsum_ref[...]
    gbc = gbc_ref[...]
    ssum = (jnp.dot(a, gsum, preferred_element_type=f32)
            + jnp.dot(b, gsum, preferred_element_type=f32))
    mu = jnp.dot(ssum / 64.0, gbc, preferred_element_type=f32)
    da = a - mu
    db = b - mu
    vsum = (jnp.dot(da * da, gsum, preferred_element_type=f32)
            + jnp.dot(db * db, gsum, preferred_element_type=f32))
    var = jnp.dot(vsum / 64.0, gbc, preferred_element_type=f32)
    rs = lax.rsqrt(var + EPS)
    va = jnp.maximum(da * rs * gl_ref[...] + bl_ref[...], 0.0)
    vb = jnp.maximum(db * rs * gh_ref[...] + bh_ref[...], 0.0)
    y4_ref[...] = (jnp.dot(va, wexl_ref[...], preferred_element_type=f32)
                   + jnp.dot(vb, wexh_ref[...], preferred_element_type=f32)
                   + bout4_ref[...])


def _post(m_lo, m_hi, r_lo, r_hi, n, brel, ln_g, ln_b, wout, bout):
    h = brel.shape[0]
    hh = h // 2
    d_out = wout.shape[1]
    np_ = _pad_rows(n)
    f32 = jnp.float32
    mlo4 = m_lo.reshape(np_ // 4, 128)
    mhi4 = m_hi.reshape(np_ // 4, 128)
    brl = jnp.tile(brel[:hh], 4).reshape(1, 128)
    brh = jnp.tile(brel[hh:], 4).reshape(1, 128)
    gl = jnp.tile(ln_g[:hh], 4).reshape(1, 128)
    gh = jnp.tile(ln_g[hh:], 4).reshape(1, 128)
    bl = jnp.tile(ln_b[:hh], 4).reshape(1, 128)
    bh = jnp.tile(ln_b[hh:], 4).reshape(1, 128)
    bout4 = jnp.tile(bout, 4).reshape(1, 4 * d_out)
    gi = jnp.arange(128, dtype=jnp.int32) // hh
    gsum = (gi[:, None] == jnp.arange(4)[None, :]).astype(f32)
    gbc = gsum.T
    eye4 = jnp.eye(4, dtype=f32)
    wexl = jnp.kron(eye4, wout[:hh, :])
    wexh = jnp.kron(eye4, wout[hh:, :])
    cspec = pl.BlockSpec((1, 128), lambda i: (0, 0))
    y4 = pl.pallas_call(
        _post_body,
        grid=(np_ // BLK,),
        in_specs=[
            pl.BlockSpec((BLK // 4, 128), lambda i: (i, 0)),
            pl.BlockSpec((BLK // 4, 128), lambda i: (i, 0)),
            pl.BlockSpec((BLK // 4, 128), lambda i: (i, 0)),
            pl.BlockSpec((BLK // 4, 128), lambda i: (i, 0)),
            cspec, cspec, cspec, cspec, cspec, cspec,
            pl.BlockSpec((128, 4), lambda i: (0, 0)),
            pl.BlockSpec((4, 128), lambda i: (0, 0)),
            pl.BlockSpec((128, 4 * d_out), lambda i: (0, 0)),
            pl.BlockSpec((128, 4 * d_out), lambda i: (0, 0)),
            pl.BlockSpec((1, 4 * d_out), lambda i: (0, 0)),
        ],
        out_specs=pl.BlockSpec((BLK // 4, 4 * d_out), lambda i: (i, 0)),
        out_shape=jax.ShapeDtypeStruct((n // 4, 4 * d_out), f32),
    )(mlo4, mhi4, r_lo, r_hi, brl, brh, gl, gh, bl, bh, gsum, gbc,
      wexl, wexh, bout4)
    return y4.reshape(n, d_out)


def kernel(x_user, x_item, edge_index_user_clicks_item, edge_index_item_rev_clicks_user,
           Win_user, Win_item, Wrel_uc, brel_uc, Wroot_uc, Wrel_iu, brel_iu, Wroot_iu,
           ln_g_user, ln_b_user, ln_g_item, ln_b_item,
           Wout_user, bout_user, Wout_item, bout_item):
    n_user = x_user.shape[0]
    n_item = x_item.shape[0]
    e_uc = edge_index_user_clicks_item.shape[1]
    e_iu = edge_index_item_rev_clicks_user.shape[1]
    np_u = _pad_rows(n_user)
    np_i = _pad_rows(n_item)

    gu_lo, gu_hi = _proj(x_user, Win_user, Wrel_uc)
    gi_lo, gi_hi = _proj(x_item, Win_item, Wrel_iu)
    ru_lo, ru_hi = _proj(x_user, Win_user, Wroot_iu)
    ri_lo, ri_hi = _proj(x_item, Win_item, Wroot_uc)

    zeros_u = jnp.zeros((np_u, 32), jnp.float32)
    zeros_i = zeros_u if np_i == np_u else jnp.zeros((np_i, 32), jnp.float32)

    seg_uc = _make_segsum(n_item, e_uc)
    seg_iu = _make_segsum(n_user, e_iu)
    def lin(a):
        return a.reshape(a.shape[0] * 4, 32)

    mi_lo, mi_hi = seg_uc(lin(gu_lo), lin(gu_hi), zeros_i, edge_index_user_clicks_item)
    mu_lo, mu_hi = seg_iu(lin(gi_lo), lin(gi_hi), zeros_u, edge_index_item_rev_clicks_user)

    y_item = _post(mi_lo, mi_hi, ri_lo, ri_hi, n_item, brel_uc,
                   ln_g_item, ln_b_item, Wout_item, bout_item)
    y_user = _post(mu_lo, mu_hi, ru_lo, ru_hi, n_user, brel_iu,
                   ln_g_user, ln_b_user, Wout_user, bout_user)
    return (y_user, y_item)


# speedup vs baseline: 1.0188x; 1.0049x over previous
import functools

import jax
import jax.numpy as jnp
from jax import lax
from jax.experimental import pallas as pl
from jax.experimental.pallas import tpu as pltpu
from jax.experimental.pallas import tpu_sc as plsc

EPS = 1e-5
BLK = 1088


def _pad_rows(n):
    return -(-n // 128) * 128


def _proj_body(x4_ref, win4_ref, wl_ref, wh_ref, lo_ref, hi_ref, w4l, w4h):
    f32 = jnp.float32

    @pl.when(pl.program_id(0) == 0)
    def _():
        win4 = win4_ref[...]
        w4l[...] = jnp.dot(win4, wl_ref[...], preferred_element_type=f32)
        w4h[...] = jnp.dot(win4, wh_ref[...], preferred_element_type=f32)

    x4 = x4_ref[...]
    lo_ref[...] = jnp.dot(x4, w4l[...], preferred_element_type=f32)
    hi_ref[...] = jnp.dot(x4, w4h[...], preferred_element_type=f32)


def _proj(x, win, w2):
    n, d_in = x.shape
    h = win.shape[1]
    hh = h // 2
    np_ = _pad_rows(n)
    f32 = jnp.float32
    x4 = x.reshape(n // 4, 4 * d_in)
    eye4 = jnp.eye(4, dtype=f32)
    win4 = jnp.kron(eye4, win)
    wl = jnp.kron(eye4, w2[:, :hh])
    wh = jnp.kron(eye4, w2[:, hh:])
    out = jax.ShapeDtypeStruct((np_ // 4, 128), f32)
    wspec = pl.BlockSpec((4 * h, 128), lambda i: (0, 0))
    return pl.pallas_call(
        _proj_body,
        grid=(np_ // BLK,),
        in_specs=[
            pl.BlockSpec((BLK // 4, 4 * d_in), lambda i: (i, 0)),
            pl.BlockSpec((4 * d_in, 4 * h), lambda i: (0, 0)),
            wspec, wspec,
        ],
        out_specs=[pl.BlockSpec((BLK // 4, 128), lambda i: (i, 0))] * 2,
        out_shape=[out] * 2,
        scratch_shapes=[pltpu.VMEM((4 * d_in, 128), f32)] * 2,
    )(x4, win4, wl, wh)


NR = 3
NI = 6


def _make_segsum(n, e):
    np_ = _pad_rows(n)
    ns = 16
    rpt = np_ // ns
    ept = e // ns
    ch = 128
    nfull = ept // ch
    tail = ept - nfull * ch
    assert np_ % (8 * ns) == 0 and e % ns == 0 and tail % 8 == 0
    assert nfull >= 3 * NI
    loop_lo = NI
    loop_hi = loop_lo + ((nfull - loop_lo - (NI - NR)) // NI) * NI
    nblocks = (loop_hi - loop_lo) // NI
    n_peel_hi = nfull - loop_hi

    mesh = plsc.VectorSubcoreMesh(core_axis_name="c", subcore_axis_name="s")
    half = jax.ShapeDtypeStruct((np_, 32), jnp.float32)
    scratch = (
        [pltpu.VMEM_SHARED((np_, 32), jnp.float32)]
        + [pltpu.VMEM((ch,), jnp.int32) for _ in range(2 * NI)]
        + [pltpu.VMEM((ch, 32), jnp.float32) for _ in range(NR)]
        + [pltpu.VMEM((tail,), jnp.int32) for _ in range(2)]
        + [pltpu.VMEM((tail, 32), jnp.float32)]
        + [pltpu.SemaphoreType.DMA for _ in range(NI + 2 * NR + 1)]
    )

    @functools.partial(
        pl.kernel,
        out_type=(half, half),
        mesh=mesh,
        scratch_types=scratch,
        compiler_params=pltpu.CompilerParams(use_tc_tiling_on_sc=False),
    )
    def seg(glo, ghi, zeros, ei, out_lo, out_hi, acc, *sc):
        sidx = sc[0:NI]
        didx = sc[NI:2 * NI]
        rows = sc[2 * NI:2 * NI + NR]
        sidxt, didxt, rowst = sc[2 * NI + NR:2 * NI + NR + 3]
        sems = sc[2 * NI + NR + 3:]
        semi = sems[0:NI]
        semg = sems[NI:NI + NR]
        semsc = sems[NI + NR:NI + 2 * NR]
        semt = sems[NI + 2 * NR]

        c = lax.axis_index("c")
        s = lax.axis_index("s")
        r0 = s * rpt
        e0 = s * ept

        def run(g, out):
            def idx_start(i, bi):
                eb = e0 + i * ch
                pltpu.async_copy(ei.at[0, pl.ds(eb, ch)], sidx[bi], semi[bi])
                pltpu.async_copy(ei.at[1, pl.ds(eb, ch)], didx[bi], semi[bi])

            def idx_wait(bi):
                pltpu.make_async_copy(ei.at[0, pl.ds(0, ch)], sidx[bi], semi[bi]).wait()
                pltpu.make_async_copy(ei.at[1, pl.ds(0, ch)], didx[bi], semi[bi]).wait()

            def g_start(br, bi):
                pltpu.async_copy(g.at[sidx[bi]], rows[br], semg[br])

            def g_wait(br, bi):
                pltpu.make_async_copy(g.at[sidx[bi]], rows[br], semg[br]).wait()

            def s_start(br, bi):
                pltpu.async_copy(rows[br], acc.at[didx[bi]], semsc[br], add=True)

            def s_wait(br, bi):
                pltpu.make_async_copy(rows[br], acc.at[didx[bi]], semsc[br]).wait()

            def visit(i, j, do_swait, do_gwait, do_idx):
                br, bi = j % NR, j % NI
                if do_swait:
                    s_wait(br, (j - NR) % NI)
                idx_wait(bi)
                g_start(br, bi)
                if do_gwait:
                    g_wait((j - 1) % NR, (j - 1) % NI)
                    s_start((j - 1) % NR, (j - 1) % NI)
                if do_idx:
                    idx_start(i + (NI - NR), (j + (NI - NR)) % NI)

            pltpu.sync_copy(zeros.at[pl.ds(r0, rpt)], acc.at[pl.ds(r0, rpt)])
            for j in range(NI - NR):
                idx_start(j, j)
            plsc.subcore_barrier()

            for j in range(NI):
                visit(j, j, do_swait=(j >= NR), do_gwait=(j >= 1), do_idx=True)

            def outer(k, carry):
                o = k * NI
                for j in range(NI):
                    visit(o + j, j, True, True, True)
                return carry

            if nblocks > 0:
                lax.fori_loop(1, 1 + nblocks, outer, 0)

            for jj in range(n_peel_hi):
                i = loop_hi + jj
                visit(i, i % NI, True, True, do_idx=(i + (NI - NR) < nfull))

            last = nfull - 1
            g_wait(last % NR, last % NI)
            s_start(last % NR, last % NI)
            if tail:
                et = e0 + nfull * ch
                pltpu.sync_copy(ei.at[0, pl.ds(et, tail)], sidxt)
                pltpu.sync_copy(ei.at[1, pl.ds(et, tail)], didxt)
                pltpu.async_copy(g.at[sidxt], rowst, semt).wait()
                pltpu.async_copy(rowst, acc.at[didxt], semt, add=True)
            for d in range(NR - 1, -1, -1):
                cb = last - d
                s_wait(cb % NR, cb % NI)
            if tail:
                pltpu.make_async_copy(rowst, acc.at[didxt], semt).wait()
            plsc.subcore_barrier()
            pltpu.sync_copy(acc.at[pl.ds(r0, rpt)], out.at[pl.ds(r0, rpt)])

        @pl.when(c == 0)
        def _():
            run(glo, out_lo)

        @pl.when(c == 1)
        def _():
            run(ghi, out_hi)

    return seg


def _post_body(mlo_ref, mhi_ref, rlo_ref, rhi_ref, br_ref, g_ref, b_ref,
               gsum_ref, gbc_ref, wex_ref, bout4_ref, y4_ref):
    f32 = jnp.float32
    m = jnp.concatenate(
        [mlo_ref[...] + rlo_ref[...], mhi_ref[...] + rhi_ref[...]], axis=1)
    m = m + br_ref[...]
    gsum = gsum_ref[...]
    gbc = gbc_ref[...]
    ssum = jnp.dot(m, gsum, preferred_element_type=f32)
    mu = jnp.dot(ssum / 64.0, gbc, preferred_element_type=f32)
    d = m - mu
    vsum = jnp.dot(d * d, gsum, preferred_element_type=f32)
    var = jnp.dot(vsum / 64.0, gbc, preferred_element_type=f32)
    v = jnp.maximum(d * lax.rsqrt(var + EPS) * g_ref[...] + b_ref[...], 0.0)
    y4_ref[...] = jnp.dot(v, wex_ref[...], preferred_element_type=f32) + bout4_ref[...]


def _post(m_lo, m_hi, r_lo, r_hi, n, brel, ln_g, ln_b, wout, bout):
    h = brel.shape[0]
    hh = h // 2
    d_out = wout.shape[1]
    np_ = _pad_rows(n)
    f32 = jnp.float32
    mlo4 = m_lo.reshape(np_ // 4, 128)
    mhi4 = m_hi.reshape(np_ // 4, 128)
    br = jnp.concatenate([jnp.tile(brel[:hh], 4), jnp.tile(brel[hh:], 4)]).reshape(1, 256)
    g = jnp.concatenate([jnp.tile(ln_g[:hh], 4), jnp.tile(ln_g[hh:], 4)]).reshape(1, 256)
    b = jnp.concatenate([jnp.tile(ln_b[:hh], 4), jnp.tile(ln_b[hh:], 4)]).reshape(1, 256)
    bout4 = jnp.tile(bout, 4).reshape(1, 4 * d_out)
    gi = (jnp.arange(256, dtype=jnp.int32) // hh) % 4
    gsum = (gi[:, None] == jnp.arange(4)[None, :]).astype(f32)
    gbc = jnp.concatenate([gsum.T[:, :128], gsum.T[:, 128:]], axis=1)
    eye4 = jnp.eye(4, dtype=f32)
    wex = jnp.concatenate([jnp.kron(eye4, wout[:hh, :]),
                           jnp.kron(eye4, wout[hh:, :])], axis=0)
    cspec = pl.BlockSpec((1, 256), lambda i: (0, 0))
    y4 = pl.pallas_call(
        _post_body,
        grid=(np_ // BLK,),
        in_specs=[
            pl.BlockSpec((BLK // 4, 128), lambda i: (i, 0)),
            pl.BlockSpec((BLK // 4, 128), lambda i: (i, 0)),
            pl.BlockSpec((BLK // 4, 128), lambda i: (i, 0)),
            pl.BlockSpec((BLK // 4, 128), lambda i: (i, 0)),
            cspec, cspec, cspec,
            pl.BlockSpec((256, 4), lambda i: (0, 0)),
            pl.BlockSpec((4, 256), lambda i: (0, 0)),
            pl.BlockSpec((256, 4 * d_out), lambda i: (0, 0)),
            pl.BlockSpec((1, 4 * d_out), lambda i: (0, 0)),
        ],
        out_specs=pl.BlockSpec((BLK // 4, 4 * d_out), lambda i: (i, 0)),
        out_shape=jax.ShapeDtypeStruct((n // 4, 4 * d_out), f32),
    )(mlo4, mhi4, r_lo, r_hi, br, g, b, gsum, gbc, wex, bout4)
    return y4.reshape(n, d_out)


def kernel(x_user, x_item, edge_index_user_clicks_item, edge_index_item_rev_clicks_user,
           Win_user, Win_item, Wrel_uc, brel_uc, Wroot_uc, Wrel_iu, brel_iu, Wroot_iu,
           ln_g_user, ln_b_user, ln_g_item, ln_b_item,
           Wout_user, bout_user, Wout_item, bout_item):
    n_user = x_user.shape[0]
    n_item = x_item.shape[0]
    e_uc = edge_index_user_clicks_item.shape[1]
    e_iu = edge_index_item_rev_clicks_user.shape[1]
    np_u = _pad_rows(n_user)
    np_i = _pad_rows(n_item)

    gu_lo, gu_hi = _proj(x_user, Win_user, Wrel_uc)
    gi_lo, gi_hi = _proj(x_item, Win_item, Wrel_iu)
    ru_lo, ru_hi = _proj(x_user, Win_user, Wroot_iu)
    ri_lo, ri_hi = _proj(x_item, Win_item, Wroot_uc)

    zeros_u = jnp.zeros((np_u, 32), jnp.float32)
    zeros_i = zeros_u if np_i == np_u else jnp.zeros((np_i, 32), jnp.float32)

    seg_uc = _make_segsum(n_item, e_uc)
    seg_iu = _make_segsum(n_user, e_iu)
    def lin(a):
        return a.reshape(a.shape[0] * 4, 32)

    mi_lo, mi_hi = seg_uc(lin(gu_lo), lin(gu_hi), zeros_i, edge_index_user_clicks_item)
    mu_lo, mu_hi = seg_iu(lin(gi_lo), lin(gi_hi), zeros_u, edge_index_item_rev_clicks_user)

    y_item = _post(mi_lo, mi_hi, ri_lo, ri_hi, n_item, brel_uc,
                   ln_g_item, ln_b_item, Wout_item, bout_item)
    y_user = _post(mu_lo, mu_hi, ru_lo, ru_hi, n_user, brel_iu,
                   ln_g_user, ln_b_user, Wout_user, bout_user)
    return (y_user, y_item)


# speedup vs baseline: 1.2384x; 1.2155x over previous
import functools

import jax
import jax.numpy as jnp
from jax import lax
from jax.experimental import pallas as pl
from jax.experimental.pallas import tpu as pltpu
from jax.experimental.pallas import tpu_sc as plsc

EPS = 1e-5
BLK = 1088


def _pad_rows(n):
    return -(-n // 128) * 128


def _proj_body(x4_ref, win4_ref, wl_ref, wh_ref, lo_ref, hi_ref, w4l, w4h):
    f32 = jnp.float32

    @pl.when(pl.program_id(0) == 0)
    def _():
        win4 = win4_ref[...]
        w4l[...] = jnp.dot(win4, wl_ref[...], preferred_element_type=f32)
        w4h[...] = jnp.dot(win4, wh_ref[...], preferred_element_type=f32)

    x4 = x4_ref[...]
    lo_ref[...] = jnp.dot(x4, w4l[...], preferred_element_type=f32)
    hi_ref[...] = jnp.dot(x4, w4h[...], preferred_element_type=f32)


def _proj(x, win, w2):
    n, d_in = x.shape
    h = win.shape[1]
    hh = h // 2
    np_ = _pad_rows(n)
    f32 = jnp.float32
    x4 = x.reshape(n // 4, 4 * d_in)
    eye4 = jnp.eye(4, dtype=f32)
    win4 = jnp.kron(eye4, win)
    wl = jnp.kron(eye4, w2[:, :hh])
    wh = jnp.kron(eye4, w2[:, hh:])
    out = jax.ShapeDtypeStruct((np_ // 4, 128), f32)
    wspec = pl.BlockSpec((4 * h, 128), lambda i: (0, 0))
    return pl.pallas_call(
        _proj_body,
        grid=(np_ // BLK,),
        in_specs=[
            pl.BlockSpec((BLK // 4, 4 * d_in), lambda i: (i, 0)),
            pl.BlockSpec((4 * d_in, 4 * h), lambda i: (0, 0)),
            wspec, wspec,
        ],
        out_specs=[pl.BlockSpec((BLK // 4, 128), lambda i: (i, 0))] * 2,
        out_shape=[out] * 2,
        scratch_shapes=[pltpu.VMEM((4 * d_in, 128), f32)] * 2,
    )(x4, win4, wl, wh)


NR = 3
NI = 6


def _make_segsum(n, e):
    np_ = _pad_rows(n)
    ns = 16
    rpt = np_ // ns
    ept = e // ns
    ch = 256
    nfull = ept // ch
    tail = ept - nfull * ch
    assert np_ % (8 * ns) == 0 and e % ns == 0 and tail % 8 == 0
    assert nfull >= 3 * NI
    loop_lo = NI
    loop_hi = loop_lo + ((nfull - loop_lo - (NI - NR)) // NI) * NI
    nblocks = (loop_hi - loop_lo) // NI
    n_peel_hi = nfull - loop_hi

    mesh = plsc.VectorSubcoreMesh(core_axis_name="c", subcore_axis_name="s")
    half = jax.ShapeDtypeStruct((np_, 32), jnp.float32)
    scratch = (
        [pltpu.VMEM_SHARED((np_, 32), jnp.float32)]
        + [pltpu.VMEM((ch,), jnp.int32) for _ in range(2 * NI)]
        + [pltpu.VMEM((ch, 32), jnp.float32) for _ in range(NR)]
        + [pltpu.VMEM((tail,), jnp.int32) for _ in range(2)]
        + [pltpu.VMEM((tail, 32), jnp.float32)]
        + [pltpu.SemaphoreType.DMA for _ in range(NI + 2 * NR + 1)]
    )

    @functools.partial(
        pl.kernel,
        out_type=(half, half),
        mesh=mesh,
        scratch_types=scratch,
        compiler_params=pltpu.CompilerParams(use_tc_tiling_on_sc=False),
    )
    def seg(glo, ghi, zeros, ei, out_lo, out_hi, acc, *sc):
        sidx = sc[0:NI]
        didx = sc[NI:2 * NI]
        rows = sc[2 * NI:2 * NI + NR]
        sidxt, didxt, rowst = sc[2 * NI + NR:2 * NI + NR + 3]
        sems = sc[2 * NI + NR + 3:]
        semi = sems[0:NI]
        semg = sems[NI:NI + NR]
        semsc = sems[NI + NR:NI + 2 * NR]
        semt = sems[NI + 2 * NR]

        c = lax.axis_index("c")
        s = lax.axis_index("s")
        r0 = s * rpt
        e0 = s * ept

        def run(g, out):
            def idx_start(i, bi):
                eb = e0 + i * ch
                pltpu.async_copy(ei.at[0, pl.ds(eb, ch)], sidx[bi], semi[bi])
                pltpu.async_copy(ei.at[1, pl.ds(eb, ch)], didx[bi], semi[bi])

            def idx_wait(bi):
                pltpu.make_async_copy(ei.at[0, pl.ds(0, ch)], sidx[bi], semi[bi]).wait()
                pltpu.make_async_copy(ei.at[1, pl.ds(0, ch)], didx[bi], semi[bi]).wait()

            def g_start(br, bi):
                pltpu.async_copy(g.at[sidx[bi]], rows[br], semg[br])

            def g_wait(br, bi):
                pltpu.make_async_copy(g.at[sidx[bi]], rows[br], semg[br]).wait()

            def s_start(br, bi):
                pltpu.async_copy(rows[br], acc.at[didx[bi]], semsc[br], add=True)

            def s_wait(br, bi):
                pltpu.make_async_copy(rows[br], acc.at[didx[bi]], semsc[br]).wait()

            def visit(i, j, do_swait, do_gwait, do_idx):
                br, bi = j % NR, j % NI
                if do_swait:
                    s_wait(br, (j - NR) % NI)
                idx_wait(bi)
                g_start(br, bi)
                if do_gwait:
                    g_wait((j - 1) % NR, (j - 1) % NI)
                    s_start((j - 1) % NR, (j - 1) % NI)
                if do_idx:
                    idx_start(i + (NI - NR), (j + (NI - NR)) % NI)

            pltpu.sync_copy(zeros.at[pl.ds(r0, rpt)], acc.at[pl.ds(r0, rpt)])
            for j in range(NI - NR):
                idx_start(j, j)
            plsc.subcore_barrier()

            for j in range(NI):
                visit(j, j, do_swait=(j >= NR), do_gwait=(j >= 1), do_idx=True)

            def outer(k, carry):
                o = k * NI
                for j in range(NI):
                    visit(o + j, j, True, True, True)
                return carry

            if nblocks > 0:
                lax.fori_loop(1, 1 + nblocks, outer, 0)

            for jj in range(n_peel_hi):
                i = loop_hi + jj
                visit(i, i % NI, True, True, do_idx=(i + (NI - NR) < nfull))

            last = nfull - 1
            g_wait(last % NR, last % NI)
            s_start(last % NR, last % NI)
            if tail:
                et = e0 + nfull * ch
                pltpu.sync_copy(ei.at[0, pl.ds(et, tail)], sidxt)
                pltpu.sync_copy(ei.at[1, pl.ds(et, tail)], didxt)
                pltpu.async_copy(g.at[sidxt], rowst, semt).wait()
                pltpu.async_copy(rowst, acc.at[didxt], semt, add=True)
            for d in range(NR - 1, -1, -1):
                cb = last - d
                s_wait(cb % NR, cb % NI)
            if tail:
                pltpu.make_async_copy(rowst, acc.at[didxt], semt).wait()
            plsc.subcore_barrier()
            pltpu.sync_copy(acc.at[pl.ds(r0, rpt)], out.at[pl.ds(r0, rpt)])

        @pl.when(c == 0)
        def _():
            run(glo, out_lo)

        @pl.when(c == 1)
        def _():
            run(ghi, out_hi)

    return seg


def _post_body(mlo_ref, mhi_ref, rlo_ref, rhi_ref, br_ref, g_ref, b_ref,
               gsum_ref, gbc_ref, wex_ref, bout4_ref, y4_ref):
    f32 = jnp.float32
    m = jnp.concatenate(
        [mlo_ref[...] + rlo_ref[...], mhi_ref[...] + rhi_ref[...]], axis=1)
    m = m + br_ref[...]
    gsum = gsum_ref[...]
    gbc = gbc_ref[...]
    ssum = jnp.dot(m, gsum, preferred_element_type=f32)
    mu = jnp.dot(ssum / 64.0, gbc, preferred_element_type=f32)
    d = m - mu
    vsum = jnp.dot(d * d, gsum, preferred_element_type=f32)
    var = jnp.dot(vsum / 64.0, gbc, preferred_element_type=f32)
    v = jnp.maximum(d * lax.rsqrt(var + EPS) * g_ref[...] + b_ref[...], 0.0)
    y4_ref[...] = jnp.dot(v, wex_ref[...], preferred_element_type=f32) + bout4_ref[...]


def _post(m_lo, m_hi, r_lo, r_hi, n, brel, ln_g, ln_b, wout, bout):
    h = brel.shape[0]
    hh = h // 2
    d_out = wout.shape[1]
    np_ = _pad_rows(n)
    f32 = jnp.float32
    mlo4 = m_lo.reshape(np_ // 4, 128)
    mhi4 = m_hi.reshape(np_ // 4, 128)
    br = jnp.concatenate([jnp.tile(brel[:hh], 4), jnp.tile(brel[hh:], 4)]).reshape(1, 256)
    g = jnp.concatenate([jnp.tile(ln_g[:hh], 4), jnp.tile(ln_g[hh:], 4)]).reshape(1, 256)
    b = jnp.concatenate([jnp.tile(ln_b[:hh], 4), jnp.tile(ln_b[hh:], 4)]).reshape(1, 256)
    bout4 = jnp.tile(bout, 4).reshape(1, 4 * d_out)
    gi = (jnp.arange(256, dtype=jnp.int32) // hh) % 4
    gsum = (gi[:, None] == jnp.arange(4)[None, :]).astype(f32)
    gbc = jnp.concatenate([gsum.T[:, :128], gsum.T[:, 128:]], axis=1)
    eye4 = jnp.eye(4, dtype=f32)
    wex = jnp.concatenate([jnp.kron(eye4, wout[:hh, :]),
                           jnp.kron(eye4, wout[hh:, :])], axis=0)
    cspec = pl.BlockSpec((1, 256), lambda i: (0, 0))
    y4 = pl.pallas_call(
        _post_body,
        grid=(np_ // BLK,),
        in_specs=[
            pl.BlockSpec((BLK // 4, 128), lambda i: (i, 0)),
            pl.BlockSpec((BLK // 4, 128), lambda i: (i, 0)),
            pl.BlockSpec((BLK // 4, 128), lambda i: (i, 0)),
            pl.BlockSpec((BLK // 4, 128), lambda i: (i, 0)),
            cspec, cspec, cspec,
            pl.BlockSpec((256, 4), lambda i: (0, 0)),
            pl.BlockSpec((4, 256), lambda i: (0, 0)),
            pl.BlockSpec((256, 4 * d_out), lambda i: (0, 0)),
            pl.BlockSpec((1, 4 * d_out), lambda i: (0, 0)),
        ],
        out_specs=pl.BlockSpec((BLK // 4, 4 * d_out), lambda i: (i, 0)),
        out_shape=jax.ShapeDtypeStruct((n // 4, 4 * d_out), f32),
    )(mlo4, mhi4, r_lo, r_hi, br, g, b, gsum, gbc, wex, bout4)
    return y4.reshape(n, d_out)


def kernel(x_user, x_item, edge_index_user_clicks_item, edge_index_item_rev_clicks_user,
           Win_user, Win_item, Wrel_uc, brel_uc, Wroot_uc, Wrel_iu, brel_iu, Wroot_iu,
           ln_g_user, ln_b_user, ln_g_item, ln_b_item,
           Wout_user, bout_user, Wout_item, bout_item):
    n_user = x_user.shape[0]
    n_item = x_item.shape[0]
    e_uc = edge_index_user_clicks_item.shape[1]
    e_iu = edge_index_item_rev_clicks_user.shape[1]
    np_u = _pad_rows(n_user)
    np_i = _pad_rows(n_item)

    gu_lo, gu_hi = _proj(x_user, Win_user, Wrel_uc)
    gi_lo, gi_hi = _proj(x_item, Win_item, Wrel_iu)
    ru_lo, ru_hi = _proj(x_user, Win_user, Wroot_iu)
    ri_lo, ri_hi = _proj(x_item, Win_item, Wroot_uc)

    zeros_u = jnp.zeros((np_u, 32), jnp.float32)
    zeros_i = zeros_u if np_i == np_u else jnp.zeros((np_i, 32), jnp.float32)

    seg_uc = _make_segsum(n_item, e_uc)
    seg_iu = _make_segsum(n_user, e_iu)
    def lin(a):
        return a.reshape(a.shape[0] * 4, 32)

    mi_lo, mi_hi = seg_uc(lin(gu_lo), lin(gu_hi), zeros_i, edge_index_user_clicks_item)
    mu_lo, mu_hi = seg_iu(lin(gi_lo), lin(gi_hi), zeros_u, edge_index_item_rev_clicks_user)

    y_item = _post(mi_lo, mi_hi, ri_lo, ri_hi, n_item, brel_uc,
                   ln_g_item, ln_b_item, Wout_item, bout_item)
    y_user = _post(mu_lo, mu_hi, ru_lo, ru_hi, n_user, brel_iu,
                   ln_g_user, ln_b_user, Wout_user, bout_user)
    return (y_user, y_item)
